# baseline (speedup 1.0000x reference)
import functools

import jax
import jax.numpy as jnp
from jax import lax
from jax.experimental import pallas as pl
from jax.experimental.pallas import tpu as pltpu
from jax.experimental.pallas import tpu_sc as plsc

_HIDDEN = 128
_MAX_NB = 6
_DEPTH = 6
_MOL_LEN = 25
_NC = 2
_NS = 16
_NW = _NC * _NS


def _gather_sum_body(n_blocks, block, table, idx3, out, idx_v, buf_v,
                     sg0, sg1, so0, so1):
    wid = lax.axis_index("s") * _NC + lax.axis_index("c")
    base = wid * n_blocks * block
    grp = block * _MAX_NB
    pltpu.sync_copy(idx3.at[wid], idx_v)

    def issue(g, s, sem):
        pltpu.async_copy(table.at[idx_v.at[g]], buf_v.at[s], sem)

    def wait_gather(s, sem):
        pltpu.make_async_copy(table.at[pl.ds(0, grp)],
                              buf_v.at[s], sem).wait()

    def wait_out(s, sem):
        pltpu.make_async_copy(buf_v.at[s, pl.ds(0, block)],
                              out.at[pl.ds(0, block)], sem).wait()

    issue(0, 0, sg0)

    def outer(o, carry):
        for b in (0, 1):
            g = 2 * o + b
            sem_g = (sg0, sg1)[b]
            sem_o = (so0, so1)[b]
            if b == 0:
                @pl.when(o > 0)
                def _():
                    wait_out(1, so1)
                issue(g + 1, 1, sg1)
            else:
                wait_out(0, so0)

                @pl.when(o < n_blocks // 2 - 1)
                def _():
                    issue(g + 1, 0, sg0)
            wait_gather(b, sem_g)

            def bond_body(k, carry2):
                for c in range(_HIDDEN // 16):
                    sl = pl.ds(c * 16, 16)
                    tot = buf_v[b, 6 * k, sl]
                    for j in range(1, _MAX_NB):
                        tot = tot + buf_v[b, 6 * k + j, sl]
                    buf_v[b, k, sl] = tot
                return carry2

            lax.fori_loop(0, block, bond_body, 0, unroll=2)
            pltpu.async_copy(buf_v.at[b, pl.ds(0, block)],
                             out.at[pl.ds(base + g * block, block)], sem_o)
        return carry

    lax.fori_loop(0, n_blocks // 2, outer, 0, unroll=False)
    wait_out(1, so1)


def _make_gather_sum(n_rows, n_blocks, block):
    n_pad = _NW * n_blocks * block
    mesh = plsc.VectorSubcoreMesh(core_axis_name="c", subcore_axis_name="s")
    body = functools.partial(_gather_sum_body, n_blocks, block)
    return pl.kernel(
        body,
        out_type=jax.ShapeDtypeStruct((n_pad, _HIDDEN), jnp.float32),
        mesh=mesh,
        scratch_types=[
            pltpu.VMEM((n_blocks, block * _MAX_NB), jnp.int32),
            pltpu.VMEM((2, block * _MAX_NB, _HIDDEN), jnp.float32),
            pltpu.SemaphoreType.DMA,
            pltpu.SemaphoreType.DMA,
            pltpu.SemaphoreType.DMA,
            pltpu.SemaphoreType.DMA,
        ],
        name=f"sc_gather_sum_{n_rows}",
    )


def _gather_sum(table, graph, n_blocks, block):
    n = graph.shape[0]
    n_pad = _NW * n_blocks * block
    n_rows = table.shape[0]
    pad = (jnp.arange((n_pad - n) * _MAX_NB, dtype=jnp.int32) * 64) % n_rows
    flat = jnp.concatenate([graph.reshape(-1), pad])
    idx3 = flat.reshape(_NW, n_blocks, block * _MAX_NB)
    out = _make_gather_sum(n, n_blocks, block)(table, idx3)
    return out


def _mm_dual_body(x_ref, w_ref, o_ref, r_ref):
    y = jnp.dot(x_ref[...], w_ref[...], preferred_element_type=jnp.float32)
    o_ref[...] = y
    r_ref[...] = jax.nn.relu(y)


def _mm_dual(x, w, blk):
    n, k = x.shape
    h = w.shape[1]
    return pl.pallas_call(
        _mm_dual_body,
        grid=(n // blk,),
        in_specs=[
            pl.BlockSpec((blk, k), lambda i: (i, 0)),
            pl.BlockSpec((k, h), lambda i: (0, 0)),
        ],
        out_specs=[pl.BlockSpec((blk, h), lambda i: (i, 0)),
                   pl.BlockSpec((blk, h), lambda i: (i, 0))],
        out_shape=[jax.ShapeDtypeStruct((n, h), jnp.float32),
                   jax.ShapeDtypeStruct((n, h), jnp.float32)],
    )(x, w)


def _update_body(b_ref, x_ref, w_ref, o_ref):
    o_ref[...] = jax.nn.relu(
        b_ref[...]
        + jnp.dot(x_ref[...], w_ref[...], preferred_element_type=jnp.float32))


def _update(binput, nei, w, blk):
    n, h = binput.shape
    return pl.pallas_call(
        _update_body,
        grid=(n // blk,),
        in_specs=[
            pl.BlockSpec((blk, h), lambda i: (i, 0)),
            pl.BlockSpec((blk, h), lambda i: (i, 0)),
            pl.BlockSpec((h, h), lambda i: (0, 0)),
        ],
        out_specs=pl.BlockSpec((blk, h), lambda i: (i, 0)),
        out_shape=jax.ShapeDtypeStruct((n, h), jnp.float32),
    )(binput, nei, w)


def _out_pool_body(mols_blk, f_ref, a_ref, wf_ref, wa_ref, b_ref, o_ref):
    atoms = mols_blk * _MOL_LEN
    h = jax.nn.relu(
        jnp.dot(f_ref[...], wf_ref[...], preferred_element_type=jnp.float32)
        + jnp.dot(a_ref[...], wa_ref[...], preferred_element_type=jnp.float32)
        + b_ref[...])
    rows = lax.broadcasted_iota(jnp.int32, (mols_blk, atoms), 0)
    cols = lax.broadcasted_iota(jnp.int32, (mols_blk, atoms), 1)
    pool = jnp.where(cols // _MOL_LEN == rows, 1.0 / _MOL_LEN, 0.0)
    o_ref[...] = jnp.dot(pool, h, preferred_element_type=jnp.float32)


def _out_pool(fatoms, anei, w_f, w_a, bias, mols_blk):
    n_atoms, fdim = fatoms.shape
    h = w_f.shape[1]
    n_mols = n_atoms // _MOL_LEN
    atoms_blk = mols_blk * _MOL_LEN
    return pl.pallas_call(
        functools.partial(_out_pool_body, mols_blk),
        grid=(n_mols // mols_blk,),
        in_specs=[
            pl.BlockSpec((atoms_blk, fdim), lambda i: (i, 0)),
            pl.BlockSpec((atoms_blk, h), lambda i: (i, 0)),
            pl.BlockSpec((fdim, h), lambda i: (0, 0)),
            pl.BlockSpec((h, h), lambda i: (0, 0)),
            pl.BlockSpec((1, h), lambda i: (0, 0)),
        ],
        out_specs=pl.BlockSpec((mols_blk, h), lambda i: (i, 0)),
        out_shape=jax.ShapeDtypeStruct((n_mols, h), jnp.float32),
    )(fatoms, anei, w_f, w_a, bias)


def kernel(fatoms, fbonds, agraph, bgraph, scope, W_i, W_h, W_o_w, W_o_b):
    fdim = fatoms.shape[1]

    binput, message = _mm_dual(fbonds, jnp.transpose(W_i), blk=10000)
    w_hT = jnp.transpose(W_h)

    for _ in range(_DEPTH - 1):
        nei = _gather_sum(message, bgraph, n_blocks=200, block=16)
        message = _update(binput, nei, w_hT, blk=10000)

    anei = _gather_sum(message, agraph, n_blocks=100, block=16)
    w_f = jnp.transpose(W_o_w[:, :fdim])
    w_a = jnp.transpose(W_o_w[:, fdim:])
    return _out_pool(fatoms, anei, w_f, w_a, W_o_b.reshape(1, -1),
                     mols_blk=200)


# speedup vs baseline: 1.7196x; 1.7196x over previous
import functools

import jax
import jax.numpy as jnp
from jax import lax
from jax.experimental import pallas as pl
from jax.experimental.pallas import tpu as pltpu
from jax.experimental.pallas import tpu_sc as plsc

_HIDDEN = 128
_MAX_NB = 6
_DEPTH = 6
_MOL_LEN = 25
_NC = 2
_NS = 16
_NW = _NC * _NS


def _gather_sum_body(n_blocks, block, table, idx4, out, idx_v, buf_v,
                     sg0, sg1, so0, so1):
    wid = lax.axis_index("s") * _NC + lax.axis_index("c")
    per_tile = n_blocks * block
    base = wid * per_tile
    pltpu.sync_copy(idx4.at[wid], idx_v)

    def issue(g, s, sem):
        for j in range(_MAX_NB):
            isl = idx_v.at[j, g // 2, pl.ds((g % 2) * block, block)]
            pltpu.async_copy(table.at[isl], buf_v.at[s, j], sem)

    def wait_gathers(s, sem):
        for j in range(_MAX_NB):
            pltpu.make_async_copy(table.at[pl.ds(0, block)],
                                  buf_v.at[s, j], sem).wait()

    def wait_out(s, sem):
        pltpu.make_async_copy(buf_v.at[s, 0],
                              out.at[pl.ds(0, block)], sem).wait()

    issue(0, 0, sg0)

    def outer(o, carry):
        for b in (0, 1):
            g = 2 * o + b
            sem_g = (sg0, sg1)[b]
            sem_o = (so0, so1)[b]
            if b == 0:
                @pl.when(o > 0)
                def _():
                    wait_out(1, so1)
                issue(g + 1, 1, sg1)
            else:
                wait_out(0, so0)

                @pl.when(o < n_blocks // 2 - 1)
                def _():
                    issue(g + 1, 0, sg0)
            wait_gathers(b, sem_g)

            def row_body(r, carry2):
                for c in range(_HIDDEN // 16):
                    sl = pl.ds(c * 16, 16)
                    tot = buf_v[b, 0, r, sl]
                    for j in range(1, _MAX_NB):
                        tot = tot + buf_v[b, j, r, sl]
                    buf_v[b, 0, r, sl] = tot
                return carry2

            lax.fori_loop(0, block, row_body, 0, unroll=2)
            pltpu.async_copy(buf_v.at[b, 0],
                             out.at[pl.ds(base + g * block, block)], sem_o)
        return carry

    lax.fori_loop(0, n_blocks // 2, outer, 0, unroll=False)
    wait_out(1, so1)


def _make_gather_sum(n_rows, n_blocks, block):
    n_pad = _NW * n_blocks * block
    mesh = plsc.VectorSubcoreMesh(core_axis_name="c", subcore_axis_name="s")
    body = functools.partial(_gather_sum_body, n_blocks, block)
    return pl.kernel(
        body,
        out_type=jax.ShapeDtypeStruct((n_pad, _HIDDEN), jnp.float32),
        mesh=mesh,
        scratch_types=[
            pltpu.VMEM((_MAX_NB, n_blocks // 2, 2 * block), jnp.int32),
            pltpu.VMEM((2, _MAX_NB, block, _HIDDEN), jnp.float32),
            pltpu.SemaphoreType.DMA,
            pltpu.SemaphoreType.DMA,
            pltpu.SemaphoreType.DMA,
            pltpu.SemaphoreType.DMA,
        ],
        name=f"sc_gather_sum_{n_rows}",
    )


def _gather_sum(table, graph, n_blocks, block):
    n = graph.shape[0]
    n_pad = _NW * n_blocks * block
    idx = jnp.transpose(graph)
    n_rows = table.shape[0]
    pad = (jnp.arange(n_pad - n, dtype=jnp.int32) * 64) % n_rows
    idx = jnp.concatenate(
        [idx, jnp.broadcast_to(pad, (_MAX_NB, n_pad - n))], axis=1)
    idx4 = jnp.transpose(idx.reshape(_MAX_NB, _NW, n_blocks, block),
                         (1, 0, 2, 3)).reshape(_NW, _MAX_NB, n_blocks // 2,
                                               2 * block)
    out = _make_gather_sum(n, n_blocks, block)(table, idx4)
    return out


def _mm_dual_body(x_ref, w_ref, o_ref, r_ref):
    y = jnp.dot(x_ref[...], w_ref[...], preferred_element_type=jnp.float32)
    o_ref[...] = y
    r_ref[...] = jax.nn.relu(y)


def _mm_dual(x, w, blk):
    n, k = x.shape
    h = w.shape[1]
    return pl.pallas_call(
        _mm_dual_body,
        grid=(n // blk,),
        in_specs=[
            pl.BlockSpec((blk, k), lambda i: (i, 0)),
            pl.BlockSpec((k, h), lambda i: (0, 0)),
        ],
        out_specs=[pl.BlockSpec((blk, h), lambda i: (i, 0)),
                   pl.BlockSpec((blk, h), lambda i: (i, 0))],
        out_shape=[jax.ShapeDtypeStruct((n, h), jnp.float32),
                   jax.ShapeDtypeStruct((n, h), jnp.float32)],
    )(x, w)


def _update_body(b_ref, x_ref, w_ref, o_ref):
    o_ref[...] = jax.nn.relu(
        b_ref[...]
        + jnp.dot(x_ref[...], w_ref[...], preferred_element_type=jnp.float32))


def _update(binput, nei, w, blk):
    n, h = binput.shape
    return pl.pallas_call(
        _update_body,
        grid=(n // blk,),
        in_specs=[
            pl.BlockSpec((blk, h), lambda i: (i, 0)),
            pl.BlockSpec((blk, h), lambda i: (i, 0)),
            pl.BlockSpec((h, h), lambda i: (0, 0)),
        ],
        out_specs=pl.BlockSpec((blk, h), lambda i: (i, 0)),
        out_shape=jax.ShapeDtypeStruct((n, h), jnp.float32),
    )(binput, nei, w)


def _out_pool_body(mols_blk, f_ref, a_ref, wf_ref, wa_ref, b_ref, o_ref):
    atoms = mols_blk * _MOL_LEN
    h = jax.nn.relu(
        jnp.dot(f_ref[...], wf_ref[...], preferred_element_type=jnp.float32)
        + jnp.dot(a_ref[...], wa_ref[...], preferred_element_type=jnp.float32)
        + b_ref[...])
    rows = lax.broadcasted_iota(jnp.int32, (mols_blk, atoms), 0)
    cols = lax.broadcasted_iota(jnp.int32, (mols_blk, atoms), 1)
    pool = jnp.where(cols // _MOL_LEN == rows, 1.0 / _MOL_LEN, 0.0)
    o_ref[...] = jnp.dot(pool, h, preferred_element_type=jnp.float32)


def _out_pool(fatoms, anei, w_f, w_a, bias, mols_blk):
    n_atoms, fdim = fatoms.shape
    h = w_f.shape[1]
    n_mols = n_atoms // _MOL_LEN
    atoms_blk = mols_blk * _MOL_LEN
    return pl.pallas_call(
        functools.partial(_out_pool_body, mols_blk),
        grid=(n_mols // mols_blk,),
        in_specs=[
            pl.BlockSpec((atoms_blk, fdim), lambda i: (i, 0)),
            pl.BlockSpec((atoms_blk, h), lambda i: (i, 0)),
            pl.BlockSpec((fdim, h), lambda i: (0, 0)),
            pl.BlockSpec((h, h), lambda i: (0, 0)),
            pl.BlockSpec((1, h), lambda i: (0, 0)),
        ],
        out_specs=pl.BlockSpec((mols_blk, h), lambda i: (i, 0)),
        out_shape=jax.ShapeDtypeStruct((n_mols, h), jnp.float32),
    )(fatoms, anei, w_f, w_a, bias)


def kernel(fatoms, fbonds, agraph, bgraph, scope, W_i, W_h, W_o_w, W_o_b):
    fdim = fatoms.shape[1]

    binput, message = _mm_dual(fbonds, jnp.transpose(W_i), blk=10000)
    w_hT = jnp.transpose(W_h)

    for _ in range(_DEPTH - 1):
        nei = _gather_sum(message, bgraph, n_blocks=50, block=64)
        message = _update(binput, nei, w_hT, blk=10000)

    anei = _gather_sum(message, agraph, n_blocks=26, block=64)
    w_f = jnp.transpose(W_o_w[:, :fdim])
    w_a = jnp.transpose(W_o_w[:, fdim:])
    return _out_pool(fatoms, anei, w_f, w_a, W_o_b.reshape(1, -1),
                     mols_blk=200)
